# Initial kernel scaffold; baseline (speedup 1.0000x reference)
import jax
import jax.numpy as jnp
from jax.experimental import pallas as pl


def kernel(x, energy_bins, emb_table):
    raise NotImplementedError("write your pallas kernel here")


# speedup vs baseline: 1.0532x; 1.0532x over previous
import jax
import jax.numpy as jnp
from jax.experimental import pallas as pl
from jax.experimental.pallas import tpu as pltpu


def _round_up(n, m):
    return ((n + m - 1) // m) * m


def _bucket_embed_body(x_ref, rep_ref, lo_ref, hi_ref, tab_ref, o_ref):
    xb = x_ref[...].astype(jnp.bfloat16)
    x_rep = jnp.dot(xb, rep_ref[...], preferred_element_type=jnp.float32)
    onehot = ((x_rep > lo_ref[...]) & (x_rep <= hi_ref[...]))
    o_ref[...] = jnp.dot(onehot.astype(jnp.bfloat16), tab_ref[...],
                         preferred_element_type=jnp.float32)


def kernel(x, energy_bins, emb_table):
    B, T = x.shape
    n_bins, D = emb_table.shape
    P = 128 // D if (0 < D < 128 and 128 % D == 0) else 1

    bins = jnp.asarray(energy_bins, jnp.float32).reshape(-1)
    lo = jnp.concatenate([jnp.full((1,), -jnp.inf, jnp.float32), bins])
    hi = jnp.concatenate([bins, jnp.full((1,), jnp.inf, jnp.float32)])
    lo_row = jnp.tile(lo, P)[None, :]
    hi_row = jnp.tile(hi, P)[None, :]
    eye_p = jnp.eye(P, dtype=jnp.float32)
    rep = jnp.kron(eye_p, jnp.ones((1, n_bins), jnp.float32))
    tab = jnp.kron(eye_p, emb_table.astype(jnp.float32))

    n = B * T
    n_pad = _round_up(n, P)
    x_flat = x.reshape(-1).astype(jnp.float32)
    if n_pad != n:
        x_flat = jnp.pad(x_flat, (0, n_pad - n))
    n_rows = n_pad // P
    x_pk = x_flat.reshape(n_rows, P)

    tile = 4096
    if n_rows < 2 * tile:
        tile = max(8, _round_up((n_rows + 1) // 2, 8))
    n_steps = -(-n_rows // tile)

    out = pl.pallas_call(
        _bucket_embed_body,
        out_shape=jax.ShapeDtypeStruct((n_rows, P * D), jnp.float32),
        grid=(n_steps,),
        in_specs=[
            pl.BlockSpec((tile, P), lambda i: (i, 0)),
            pl.BlockSpec((P, P * n_bins), lambda i: (0, 0)),
            pl.BlockSpec((1, P * n_bins), lambda i: (0, 0)),
            pl.BlockSpec((1, P * n_bins), lambda i: (0, 0)),
            pl.BlockSpec((P * n_bins, P * D), lambda i: (0, 0)),
        ],
        out_specs=pl.BlockSpec((tile, P * D), lambda i: (i, 0)),
        compiler_params=pltpu.CompilerParams(
            dimension_semantics=("parallel",)),
    )(x_pk, rep.astype(jnp.bfloat16), lo_row, hi_row, tab.astype(jnp.bfloat16))

    out = out.reshape(n_pad, D)
    if n_pad != n:
        out = out[:n]
    return out.reshape(B, T, D)


# speedup vs baseline: 1.4255x; 1.3535x over previous
import jax
import jax.numpy as jnp
from jax.experimental import pallas as pl
from jax.experimental.pallas import tpu as pltpu


def _round_up(n, m):
    return ((n + m - 1) // m) * m


def _make_body(tile, p, d):
    def body(x_ref, rep_ref, lo_ref, hi_ref, tab_ref, o_ref):
        xb = x_ref[...].astype(jnp.bfloat16)
        x_rep = jnp.dot(xb, rep_ref[...], preferred_element_type=jnp.float32)
        onehot = (x_rep > lo_ref[...]) & (x_rep <= hi_ref[...])
        dense = jnp.dot(onehot.astype(jnp.bfloat16), tab_ref[...],
                        preferred_element_type=jnp.float32)
        for k in range(p):
            o_ref[k:p * tile:p, :] = dense[:, k * d:(k + 1) * d]
    return body


def kernel(x, energy_bins, emb_table):
    B, T = x.shape
    n_bins, D = emb_table.shape
    P = 128 // D if (0 < D < 128 and 128 % D == 0) else 1

    bins = jnp.asarray(energy_bins, jnp.float32).reshape(-1)
    lo = jnp.concatenate([jnp.full((1,), -jnp.inf, jnp.float32), bins])
    hi = jnp.concatenate([bins, jnp.full((1,), jnp.inf, jnp.float32)])
    lo_row = jnp.tile(lo, P)[None, :]
    hi_row = jnp.tile(hi, P)[None, :]
    eye_p = jnp.eye(P, dtype=jnp.float32)
    rep = jnp.kron(eye_p, jnp.ones((1, n_bins), jnp.float32))
    tab = jnp.kron(eye_p, emb_table.astype(jnp.float32))

    n = B * T
    n_pad = _round_up(n, P)
    x_flat = x.reshape(-1).astype(jnp.float32)
    if n_pad != n:
        x_flat = jnp.pad(x_flat, (0, n_pad - n))
    n_rows = n_pad // P
    x_pk = x_flat.reshape(n_rows, P)

    tile = 2048
    if n_rows < 2 * tile:
        tile = max(8, _round_up((n_rows + 1) // 2, 8))
    n_steps = -(-n_rows // tile)

    out = pl.pallas_call(
        _make_body(tile, P, D),
        out_shape=jax.ShapeDtypeStruct((n_pad, D), jnp.float32),
        grid=(n_steps,),
        in_specs=[
            pl.BlockSpec((tile, P), lambda i: (i, 0)),
            pl.BlockSpec((P, P * n_bins), lambda i: (0, 0)),
            pl.BlockSpec((1, P * n_bins), lambda i: (0, 0)),
            pl.BlockSpec((1, P * n_bins), lambda i: (0, 0)),
            pl.BlockSpec((P * n_bins, P * D), lambda i: (0, 0)),
        ],
        out_specs=pl.BlockSpec((P * tile, D), lambda i: (i, 0)),
        compiler_params=pltpu.CompilerParams(
            dimension_semantics=("parallel",)),
    )(x_pk, rep.astype(jnp.bfloat16), lo_row, hi_row, tab.astype(jnp.bfloat16))

    if n_pad != n:
        out = out[:n]
    return out.reshape(B, T, D)


# speedup vs baseline: 1.4290x; 1.0025x over previous
import jax
import jax.numpy as jnp
from jax.experimental import pallas as pl
from jax.experimental.pallas import tpu as pltpu


def _round_up(n, m):
    return ((n + m - 1) // m) * m


def _make_body(tile, p, d):
    def body(x_ref, rep_ref, lo_ref, hi_ref, tab_ref, o_ref):
        xb = x_ref[...].astype(jnp.bfloat16)
        x_rep = jax.lax.dot_general(
            xb, rep_ref[...], (((0,), (0,)), ((), ())),
            preferred_element_type=jnp.float32)
        onehot = (x_rep > lo_ref[...]) & (x_rep <= hi_ref[...])
        dense = jnp.dot(onehot.astype(jnp.bfloat16), tab_ref[...],
                        preferred_element_type=jnp.float32)
        for k in range(p):
            o_ref[k:p * tile:p, :] = dense[:, k * d:(k + 1) * d]
    return body


def kernel(x, energy_bins, emb_table):
    B, T = x.shape
    n_bins, D = emb_table.shape
    P = 128 // D if (0 < D < 128 and 128 % D == 0) else 1

    bins = jnp.asarray(energy_bins, jnp.float32).reshape(-1)
    lo = jnp.concatenate([jnp.full((1,), -jnp.inf, jnp.float32), bins])
    hi = jnp.concatenate([bins, jnp.full((1,), jnp.inf, jnp.float32)])
    lo_row = jnp.tile(lo, P)[None, :]
    hi_row = jnp.tile(hi, P)[None, :]
    eye_p = jnp.eye(P, dtype=jnp.float32)
    rep = jnp.kron(eye_p, jnp.ones((1, n_bins), jnp.float32))
    tab = jnp.kron(eye_p, emb_table.astype(jnp.float32))

    n = B * T
    n_pad = _round_up(n, P)
    x_flat = x.reshape(-1).astype(jnp.float32)
    if n_pad != n:
        x_flat = jnp.pad(x_flat, (0, n_pad - n))
    n_rows = n_pad // P
    x_t = x_flat.reshape(n_rows, P).T

    tile = 2048
    if n_rows < 2 * tile:
        tile = max(8, _round_up((n_rows + 1) // 2, 8))
    n_steps = -(-n_rows // tile)

    out = pl.pallas_call(
        _make_body(tile, P, D),
        out_shape=jax.ShapeDtypeStruct((n_pad, D), jnp.float32),
        grid=(n_steps,),
        in_specs=[
            pl.BlockSpec((P, tile), lambda i: (0, i)),
            pl.BlockSpec((P, P * n_bins), lambda i: (0, 0)),
            pl.BlockSpec((1, P * n_bins), lambda i: (0, 0)),
            pl.BlockSpec((1, P * n_bins), lambda i: (0, 0)),
            pl.BlockSpec((P * n_bins, P * D), lambda i: (0, 0)),
        ],
        out_specs=pl.BlockSpec((P * tile, D), lambda i: (i, 0)),
        compiler_params=pltpu.CompilerParams(
            dimension_semantics=("parallel",)),
    )(x_t, rep.astype(jnp.bfloat16), lo_row, hi_row, tab.astype(jnp.bfloat16))

    if n_pad != n:
        out = out[:n]
    return out.reshape(B, T, D)


# speedup vs baseline: 19.1779x; 13.4205x over previous
import jax
import jax.numpy as jnp
from jax.experimental import pallas as pl
from jax.experimental.pallas import tpu as pltpu


def _make_body(bb, nb, d, tt):
    def body(x_ref, lo_ref, hi_ref, tabt_ref, o_ref):
        xb = x_ref[...].astype(jnp.bfloat16).astype(jnp.float32)
        tabt = tabt_ref[...]
        for i in range(bb):
            xrow = jax.lax.broadcast_in_dim(xb[i, :], (nb, tt), (1,))
            onehot = (xrow > lo_ref[...]) & (xrow <= hi_ref[...])
            o_ref[i] = jnp.dot(tabt, onehot.astype(jnp.bfloat16),
                               preferred_element_type=jnp.float32)
    return body


def kernel(x, energy_bins, emb_table):
    B, T = x.shape
    n_bins, D = emb_table.shape

    bins = jnp.asarray(energy_bins, jnp.float32).reshape(-1)
    lo = jnp.concatenate([jnp.full((1,), -jnp.inf, jnp.float32), bins])
    hi = jnp.concatenate([bins, jnp.full((1,), jnp.inf, jnp.float32)])
    lo_col = lo[:, None]
    hi_col = hi[:, None]
    tabt = emb_table.astype(jnp.float32).T.astype(jnp.bfloat16)

    bb = 8
    while B % bb:
        bb //= 2
    n_steps = B // bb

    out_t = pl.pallas_call(
        _make_body(bb, n_bins, D, T),
        out_shape=jax.ShapeDtypeStruct((B, D, T), jnp.float32),
        grid=(n_steps,),
        in_specs=[
            pl.BlockSpec((bb, T), lambda i: (i, 0)),
            pl.BlockSpec((n_bins, 1), lambda i: (0, 0)),
            pl.BlockSpec((n_bins, 1), lambda i: (0, 0)),
            pl.BlockSpec((D, n_bins), lambda i: (0, 0)),
        ],
        out_specs=pl.BlockSpec((bb, D, T), lambda i: (i, 0, 0)),
        compiler_params=pltpu.CompilerParams(
            dimension_semantics=("parallel",)),
    )(x, lo_col, hi_col, tabt)

    return jnp.transpose(out_t, (0, 2, 1))
